# baseline (speedup 1.0000x reference)
import jax
import jax.numpy as jnp
from jax.experimental import pallas as pl

_RADIUS2 = 0.25
_B = 2
_N = 4096
_TM = 512
_W = 32
_L = 128


def _adj_kernel(pi_ref, pjt_ref, out_ref):
    pi = pi_ref[0]
    pjt = pjt_ref[0]
    si = jnp.sum(pi * pi, axis=1, keepdims=True)
    sj = jnp.sum(pjt * pjt, axis=0, keepdims=True)
    m2 = jax.lax.dot_general(
        -2.0 * pi, pjt, (((1,), (0,)), ((), ())),
        preferred_element_type=jnp.float32)
    dist = (m2 + si) + sj
    acc = jnp.zeros((_TM, _L), jnp.int32)
    for k in range(_W):
        sl = dist[:, k * _L:(k + 1) * _L]
        bit = (1 << k) if k < 31 else -(1 << 31)
        acc = acc + jnp.where(sl <= _RADIUS2, jnp.int32(bit), jnp.int32(0))
    out_ref[0] = acc


def kernel(batch_points, batch_feats, batch_len):
    pts = batch_points.reshape(_B, _N, 3)
    fts = batch_feats.reshape(_B, _N, batch_feats.shape[-1])
    pts_t = jnp.swapaxes(pts, 1, 2)

    packed = pl.pallas_call(
        _adj_kernel,
        grid=(_B, _N // _TM),
        in_specs=[
            pl.BlockSpec((1, _TM, 3), lambda b, i: (b, i, 0)),
            pl.BlockSpec((1, 3, _N), lambda b, i: (b, 0, 0)),
        ],
        out_specs=pl.BlockSpec((1, _TM, _L), lambda b, i: (b, i, 0)),
        out_shape=jax.ShapeDtypeStruct((_B, _N, _L), jnp.int32),
    )(pts, pts_t)
    ks = jnp.arange(_W, dtype=jnp.int32).reshape(1, 1, _W, 1)
    bits = jnp.bitwise_and(
        jax.lax.shift_right_logical(packed[:, :, None, :], ks), 1)
    adj = bits.reshape(_B, _N, _N) != 0
    return adj, pts, fts


# speedup vs baseline: 3.7837x; 3.7837x over previous
import jax
import jax.numpy as jnp
from jax.experimental import pallas as pl
from jax.experimental.pallas import tpu as pltpu

_RADIUS2 = 0.25
_B = 2
_N = 4096
_TM = 512
_T = _N // _TM


def _adj_kernel(pi_ref, pjt_ref, out_ref, mir_ref):
    I = pl.program_id(1)
    pi = pi_ref[0]
    pjt = pjt_ref[0]
    si = jnp.sum(pi * pi, axis=1, keepdims=True)
    sj_full = jnp.sum(pjt * pjt, axis=0, keepdims=True)
    npi = -2.0 * pi
    for J in range(_T):
        lo = J * _TM
        @pl.when(J < I)
        def _emit_mirror():
            out_ref[0, :, lo:lo + _TM] = mir_ref[I, :, lo:lo + _TM]

        @pl.when(J >= I)
        def _compute_tile():
            m2 = jax.lax.dot_general(
                npi, pjt_ref[0, :, lo:lo + _TM], (((1,), (0,)), ((), ())),
                preferred_element_type=jnp.float32)
            dist = (m2 + si) + sj_full[:, lo:lo + _TM]
            v = (dist <= _RADIUS2).astype(jnp.int8)
            out_ref[0, :, lo:lo + _TM] = v
            @pl.when(J > I)
            def _park_mirror():
                mir_ref[J, :, pl.ds(I * _TM, _TM)] = v.T


def kernel(batch_points, batch_feats, batch_len):
    pts = batch_points.reshape(_B, _N, 3)
    fts = batch_feats.reshape(_B, _N, batch_feats.shape[-1])
    pts_t = jnp.swapaxes(pts, 1, 2)

    adj8 = pl.pallas_call(
        _adj_kernel,
        grid=(_B, _T),
        in_specs=[
            pl.BlockSpec((1, _TM, 3), lambda b, i: (b, i, 0)),
            pl.BlockSpec((1, 3, _N), lambda b, i: (b, 0, 0)),
        ],
        out_specs=pl.BlockSpec((1, _TM, _N), lambda b, i: (b, i, 0)),
        out_shape=jax.ShapeDtypeStruct((_B, _N, _N), jnp.int8),
        scratch_shapes=[pltpu.VMEM((_T, _TM, _N), jnp.int8)],
    )(pts, pts_t)
    adj = adj8.view(jnp.bool_)
    return adj, pts, fts


# speedup vs baseline: 4.9864x; 1.3178x over previous
import jax
import jax.numpy as jnp
from jax.experimental import pallas as pl
from jax.experimental.pallas import tpu as pltpu

_RADIUS2 = 0.25
_B = 2
_N = 4096
_TM = 512
_T = _N // _TM


def _row_copy(rows_ref, out_ref, sem, b, i):
    return pltpu.make_async_copy(
        rows_ref.at[b, i],
        out_ref.at[b, pl.ds(i * _TM, _TM), :],
        sem.at[b, i],
    )


def _adj_kernel(p_ref, pt_ref, out_ref, rows_ref, sem):
    for b in range(_B):
        pt = pt_ref[b]
        sj_full = jnp.sum(pt * pt, axis=0, keepdims=True)
        for I in range(_T):
            pi = p_ref[b, I * _TM:(I + 1) * _TM, :]
            si = jnp.sum(pi * pi, axis=1, keepdims=True)
            npi = -2.0 * pi
            for J in range(I, _T):
                lo = J * _TM
                m2 = jax.lax.dot_general(
                    npi, pt[:, lo:lo + _TM], (((1,), (0,)), ((), ())),
                    preferred_element_type=jnp.float32)
                dist = (m2 + si) + sj_full[:, lo:lo + _TM]
                v = (dist <= _RADIUS2).astype(jnp.int8)
                rows_ref[b, I, :, lo:lo + _TM] = v
                if J != I:
                    rows_ref[b, J, :, I * _TM:(I + 1) * _TM] = v.T
            _row_copy(rows_ref, out_ref, sem, b, I).start()
    for b in range(_B):
        for I in range(_T):
            _row_copy(rows_ref, out_ref, sem, b, I).wait()


def kernel(batch_points, batch_feats, batch_len):
    pts = batch_points.reshape(_B, _N, 3)
    fts = batch_feats.reshape(_B, _N, batch_feats.shape[-1])
    pts_t = jnp.swapaxes(pts, 1, 2)

    adj8 = pl.pallas_call(
        _adj_kernel,
        in_specs=[
            pl.BlockSpec(memory_space=pltpu.MemorySpace.VMEM),
            pl.BlockSpec(memory_space=pltpu.MemorySpace.VMEM),
        ],
        out_specs=pl.BlockSpec(memory_space=pltpu.MemorySpace.HBM),
        out_shape=jax.ShapeDtypeStruct((_B, _N, _N), jnp.int8),
        scratch_shapes=[
            pltpu.VMEM((_B, _T, _TM, _N), jnp.int8),
            pltpu.SemaphoreType.DMA((_B, _T)),
        ],
    )(pts, pts_t)
    adj = adj8.view(jnp.bool_)
    return adj, pts, fts
